# baseline (speedup 1.0000x reference)
import functools

import jax
import jax.numpy as jnp
from jax import lax
from jax.experimental import pallas as pl
from jax.experimental.pallas import tpu as pltpu
from jax.experimental.pallas import tpu_sc as plsc

N_NODES = 10000
D = 128
DW = D // 2
N_EDGES = 320000
NC = 2
NS = 16
NW = NC * NS
EPW = N_EDGES // NW
C = 80
NCHUNK = EPW // C
L = 16
GROUPS = C // L
PAIRS = (NCHUNK - 1) // 2


def _mesh():
    return plsc.VectorSubcoreMesh(core_axis_name="c", subcore_axis_name="s")


@functools.partial(
    pl.kernel,
    out_type=jax.ShapeDtypeStruct((N_EDGES,), jnp.float32),
    mesh=_mesh(),
    compiler_params=pltpu.CompilerParams(
        needs_layout_passes=False, use_tc_tiling_on_sc=False),
    scratch_types=[
        pltpu.VMEM_SHARED((N_NODES, DW), jnp.int32),
        pltpu.VMEM((EPW,), jnp.int32),
        pltpu.VMEM((EPW,), jnp.int32),
        pltpu.VMEM((C, DW), jnp.int32),
        pltpu.VMEM((C, DW), jnp.int32),
        pltpu.VMEM((C, DW), jnp.int32),
        pltpu.VMEM((C, DW), jnp.int32),
        pltpu.VMEM((L * L,), jnp.float32),
        pltpu.VMEM((EPW,), jnp.float32),
        pltpu.SemaphoreType.DMA,
        pltpu.SemaphoreType.DMA,
        pltpu.SemaphoreType.DMA,
        pltpu.SemaphoreType.DMA,
    ],
)
def _edge_dot(x_hbm, ia_hbm, ib_hbm, out_hbm,
              xs, ia_v, ib_v, ra0, rb0, ra1, rb1, part, out_v,
              sa0, sb0, sa1, sb1):
    sid = lax.axis_index("s")
    wid = sid * NC + lax.axis_index("c")
    lanes = lax.iota(jnp.int32, L)
    cols = lanes * L

    rows_per_tile = N_NODES // NS
    seg = pl.ds(sid * rows_per_tile, rows_per_tile)
    pltpu.sync_copy(x_hbm.at[seg], xs.at[seg])

    def compute(ra, rb, c):
        nk = DW // L

        def edge_dot16(e):
            prods = []
            for k in range(nk):
                pa = plsc.bitcast(ra[e, pl.ds(k * L, L)], jnp.bfloat16)
                pb = plsc.bitcast(rb[e, pl.ds(k * L, L)], jnp.bfloat16)
                prods.append(pa * pb)
            parts = []
            for p in prods:
                p0, p1 = plsc.unpack(p, format=plsc.PackFormat.INTERLEAVED)
                parts.append(p0 + p1)
            return (parts[0] + parts[1]) + (parts[2] + parts[3])

        def group(g, _):
            sums = [edge_dot16(g * L + i) for i in range(L)]
            for i, s in enumerate(sums):
                part[pl.ds(i * L, L)] = s
            tot = plsc.load_gather(part, [cols])
            for j in range(1, L):
                tot = tot + plsc.load_gather(part, [cols + j])
            out_v[pl.ds(c * C + g * L, L)] = tot
            return 0

        for g in range(GROUPS):
            group(g, 0)

    def issue(c, ra, rb, sa, sb):
        pltpu.async_copy(xs.at[ia_v.at[pl.ds(c * C, C)]], ra, sa)
        pltpu.async_copy(xs.at[ib_v.at[pl.ds(c * C, C)]], rb, sb)

    def wait(c, ra, rb, sa, sb):
        pltpu.make_async_copy(xs.at[ia_v.at[pl.ds(c * C, C)]], ra, sa).wait()
        pltpu.make_async_copy(xs.at[ib_v.at[pl.ds(c * C, C)]], rb, sb).wait()

    ebase = wid * EPW
    pltpu.sync_copy(ia_hbm.at[pl.ds(ebase, EPW)], ia_v)
    pltpu.sync_copy(ib_hbm.at[pl.ds(ebase, EPW)], ib_v)
    plsc.subcore_barrier()
    issue(0, ra0, rb0, sa0, sb0)

    def pair(i, _):
        c = 2 * i
        issue(c + 1, ra1, rb1, sa1, sb1)
        wait(c, ra0, rb0, sa0, sb0)
        compute(ra0, rb0, c)
        issue(c + 2, ra0, rb0, sa0, sb0)
        wait(c + 1, ra1, rb1, sa1, sb1)
        compute(ra1, rb1, c + 1)
        return 0

    lax.fori_loop(0, PAIRS, pair, 0)
    wait(NCHUNK - 1, ra0, rb0, sa0, sb0)
    compute(ra0, rb0, NCHUNK - 1)
    pltpu.sync_copy(out_v, out_hbm.at[pl.ds(ebase, EPW)])


def kernel(x_author, edge_label_index):
    xb = x_author.astype(jnp.bfloat16).reshape(N_NODES, DW, 2)
    x32 = jax.lax.bitcast_convert_type(xb, jnp.int32)
    idx = edge_label_index.astype(jnp.int32)
    return _edge_dot(x32, idx[0], idx[1])


# speedup vs baseline: 1.6616x; 1.6616x over previous
import functools

import jax
import jax.numpy as jnp
from jax import lax
from jax.experimental import pallas as pl
from jax.experimental.pallas import tpu as pltpu
from jax.experimental.pallas import tpu_sc as plsc

N_NODES = 10000
D = 128
DW = D // 2
N_EDGES = 320000
NC = 2
NS = 16
NW = NC * NS
EPW = N_EDGES // NW
C = 80
NCHUNK = EPW // C
L = 16
GROUPS = C // L
PAIRS = (NCHUNK - 1) // 2


def _mesh():
    return plsc.VectorSubcoreMesh(core_axis_name="c", subcore_axis_name="s")


@functools.partial(
    pl.kernel,
    out_type=jax.ShapeDtypeStruct((N_EDGES,), jnp.float32),
    mesh=_mesh(),
    compiler_params=pltpu.CompilerParams(
        needs_layout_passes=False, use_tc_tiling_on_sc=False),
    scratch_types=[
        pltpu.VMEM_SHARED((N_NODES, DW), jnp.int32),
        pltpu.VMEM((125, D), jnp.float32),
        pltpu.VMEM((125, DW), jnp.int32),
        pltpu.VMEM((EPW,), jnp.int32),
        pltpu.VMEM((EPW,), jnp.int32),
        pltpu.VMEM((C, DW), jnp.int32),
        pltpu.VMEM((C, DW), jnp.int32),
        pltpu.VMEM((C, DW), jnp.int32),
        pltpu.VMEM((C, DW), jnp.int32),
        pltpu.VMEM((L * L,), jnp.float32),
        pltpu.VMEM((EPW,), jnp.float32),
        pltpu.SemaphoreType.DMA,
        pltpu.SemaphoreType.DMA,
        pltpu.SemaphoreType.DMA,
        pltpu.SemaphoreType.DMA,
    ],
)
def _edge_dot(x_hbm, ia_hbm, ib_hbm, out_hbm,
              xs, stage_v, pk_v, ia_v, ib_v, ra0, rb0, ra1, rb1, part, out_v,
              sa0, sb0, sa1, sb1):
    sid = lax.axis_index("s")
    wid = sid * NC + lax.axis_index("c")
    lanes = lax.iota(jnp.int32, L)
    cols = lanes * L

    rows_per_tile = N_NODES // NS
    rows_per_batch = 125

    def conv_row(r, _):
        for k in range(DW // L):
            a = stage_v[r, pl.ds(2 * k * L, L)]
            b = stage_v[r, pl.ds((2 * k + 1) * L, L)]
            pk_v[r, pl.ds(k * L, L)] = plsc.bitcast(
                plsc.pack(a, b, format=plsc.PackFormat.INTERLEAVED),
                jnp.int32)
        return 0

    def stage_batch(bi, _):
        row0 = sid * rows_per_tile + bi * rows_per_batch
        pltpu.sync_copy(x_hbm.at[pl.ds(row0, rows_per_batch)], stage_v)
        lax.fori_loop(0, rows_per_batch, conv_row, 0)
        pltpu.sync_copy(pk_v, xs.at[pl.ds(row0, rows_per_batch)])
        return 0

    lax.fori_loop(0, rows_per_tile // rows_per_batch, stage_batch, 0)

    def compute(ra, rb, c):
        nk = DW // L

        def edge_dot16(e):
            prods = []
            for k in range(nk):
                pa = plsc.bitcast(ra[e, pl.ds(k * L, L)], jnp.bfloat16)
                pb = plsc.bitcast(rb[e, pl.ds(k * L, L)], jnp.bfloat16)
                prods.append(pa * pb)
            parts = []
            for p in prods:
                p0, p1 = plsc.unpack(p, format=plsc.PackFormat.INTERLEAVED)
                parts.append(p0 + p1)
            return (parts[0] + parts[1]) + (parts[2] + parts[3])

        def group(g, _):
            sums = [edge_dot16(g * L + i) for i in range(L)]
            for i, s in enumerate(sums):
                part[pl.ds(i * L, L)] = s
            tot = plsc.load_gather(part, [cols])
            for j in range(1, L):
                tot = tot + plsc.load_gather(part, [cols + j])
            out_v[pl.ds(c * C + g * L, L)] = tot
            return 0

        lax.fori_loop(0, GROUPS, group, 0)

    def issue(c, ra, rb, sa, sb):
        pltpu.async_copy(xs.at[ia_v.at[pl.ds(c * C, C)]], ra, sa)
        pltpu.async_copy(xs.at[ib_v.at[pl.ds(c * C, C)]], rb, sb)

    def wait(c, ra, rb, sa, sb):
        pltpu.make_async_copy(xs.at[ia_v.at[pl.ds(c * C, C)]], ra, sa).wait()
        pltpu.make_async_copy(xs.at[ib_v.at[pl.ds(c * C, C)]], rb, sb).wait()

    ebase = wid * EPW
    pltpu.sync_copy(ia_hbm.at[pl.ds(ebase, EPW)], ia_v)
    pltpu.sync_copy(ib_hbm.at[pl.ds(ebase, EPW)], ib_v)
    plsc.subcore_barrier()
    issue(0, ra0, rb0, sa0, sb0)

    def pair(i, _):
        c = 2 * i
        issue(c + 1, ra1, rb1, sa1, sb1)
        wait(c, ra0, rb0, sa0, sb0)
        compute(ra0, rb0, c)
        issue(c + 2, ra0, rb0, sa0, sb0)
        wait(c + 1, ra1, rb1, sa1, sb1)
        compute(ra1, rb1, c + 1)
        return 0

    lax.fori_loop(0, PAIRS, pair, 0)
    wait(NCHUNK - 1, ra0, rb0, sa0, sb0)
    compute(ra0, rb0, NCHUNK - 1)
    pltpu.sync_copy(out_v, out_hbm.at[pl.ds(ebase, EPW)])


def kernel(x_author, edge_label_index):
    idx = edge_label_index.astype(jnp.int32)
    return _edge_dot(x_author, idx[0], idx[1])


# speedup vs baseline: 1.7273x; 1.0395x over previous
import functools

import jax
import jax.numpy as jnp
from jax import lax
from jax.experimental import pallas as pl
from jax.experimental.pallas import tpu as pltpu
from jax.experimental.pallas import tpu_sc as plsc

N_NODES = 10000
D = 128
DW = D // 2
N_EDGES = 320000
NC = 2
NS = 16
NW = NC * NS
EPW = N_EDGES // NW
C = 80
NCHUNK = EPW // C
L = 16
GROUPS = C // L
PAIRS = (NCHUNK - 1) // 2


def _mesh():
    return plsc.VectorSubcoreMesh(core_axis_name="c", subcore_axis_name="s")


@functools.partial(
    pl.kernel,
    out_type=jax.ShapeDtypeStruct((N_EDGES,), jnp.float32),
    mesh=_mesh(),
    compiler_params=pltpu.CompilerParams(
        needs_layout_passes=False, use_tc_tiling_on_sc=False),
    scratch_types=[
        pltpu.VMEM_SHARED((N_NODES, DW), jnp.int32),
        pltpu.VMEM((125, D), jnp.float32),
        pltpu.VMEM((125, D), jnp.float32),
        pltpu.VMEM((125, DW), jnp.int32),
        pltpu.VMEM((EPW,), jnp.int32),
        pltpu.VMEM((EPW,), jnp.int32),
        pltpu.VMEM((C, DW), jnp.int32),
        pltpu.VMEM((C, DW), jnp.int32),
        pltpu.VMEM((C, DW), jnp.int32),
        pltpu.VMEM((C, DW), jnp.int32),
        pltpu.VMEM((L * L,), jnp.float32),
        pltpu.VMEM((EPW,), jnp.float32),
        pltpu.SemaphoreType.DMA,
        pltpu.SemaphoreType.DMA,
        pltpu.SemaphoreType.DMA,
        pltpu.SemaphoreType.DMA,
        pltpu.SemaphoreType.DMA,
        pltpu.SemaphoreType.DMA,
    ],
)
def _edge_dot(x_hbm, ia_hbm, ib_hbm, out_hbm,
              xs, sv0, sv1, pk_v, ia_v, ib_v, ra0, rb0, ra1, rb1, part, out_v,
              sa0, sb0, sa1, sb1, ss0, ss1):
    sid = lax.axis_index("s")
    wid = sid * NC + lax.axis_index("c")
    lanes = lax.iota(jnp.int32, L)
    cols = lanes * L
    ebase = wid * EPW

    pltpu.async_copy(ia_hbm.at[pl.ds(ebase, EPW)], ia_v, sa1)
    pltpu.async_copy(ib_hbm.at[pl.ds(ebase, EPW)], ib_v, sb1)

    rows_per_tile = N_NODES // NS
    rpb = 125
    nbatch = rows_per_tile // rpb
    svs = [sv0, sv1]
    sss = [ss0, ss1]

    def brows(b):
        return pl.ds(sid * rows_per_tile + b * rpb, rpb)

    def conv5(sv):
        def body(it, _):
            for j in range(5):
                r = it * 5 + j
                for k in range(DW // L):
                    a = sv[r, pl.ds(2 * k * L, L)]
                    b = sv[r, pl.ds((2 * k + 1) * L, L)]
                    pk_v[r, pl.ds(k * L, L)] = plsc.bitcast(
                        plsc.pack(a, b, format=plsc.PackFormat.INTERLEAVED),
                        jnp.int32)
            return 0
        lax.fori_loop(0, rpb // 5, body, 0)

    pltpu.async_copy(x_hbm.at[brows(0)], sv0, ss0)
    for b in range(nbatch):
        if b + 1 < nbatch:
            pltpu.async_copy(x_hbm.at[brows(b + 1)], svs[(b + 1) % 2],
                             sss[(b + 1) % 2])
        pltpu.make_async_copy(x_hbm.at[brows(b)], svs[b % 2],
                              sss[b % 2]).wait()
        conv5(svs[b % 2])
        pltpu.sync_copy(pk_v, xs.at[brows(b)])

    def compute(ra, rb, c):
        nk = DW // L

        def edge_dot16(e):
            prods = []
            for k in range(nk):
                pa = plsc.bitcast(ra[e, pl.ds(k * L, L)], jnp.bfloat16)
                pb = plsc.bitcast(rb[e, pl.ds(k * L, L)], jnp.bfloat16)
                prods.append(pa * pb)
            parts = []
            for p in prods:
                p0, p1 = plsc.unpack(p, format=plsc.PackFormat.INTERLEAVED)
                parts.append(p0 + p1)
            return (parts[0] + parts[1]) + (parts[2] + parts[3])

        def group(g, _):
            sums = [edge_dot16(g * L + i) for i in range(L)]
            for i, s in enumerate(sums):
                part[pl.ds(i * L, L)] = s
            tot = plsc.load_gather(part, [cols])
            for j in range(1, L):
                tot = tot + plsc.load_gather(part, [cols + j])
            out_v[pl.ds(c * C + g * L, L)] = tot
            return 0

        lax.fori_loop(0, GROUPS, group, 0)

    def issue(c, ra, rb, sa, sb):
        pltpu.async_copy(xs.at[ia_v.at[pl.ds(c * C, C)]], ra, sa)
        pltpu.async_copy(xs.at[ib_v.at[pl.ds(c * C, C)]], rb, sb)

    def wait(c, ra, rb, sa, sb):
        pltpu.make_async_copy(xs.at[ia_v.at[pl.ds(c * C, C)]], ra, sa).wait()
        pltpu.make_async_copy(xs.at[ib_v.at[pl.ds(c * C, C)]], rb, sb).wait()

    pltpu.make_async_copy(ia_hbm.at[pl.ds(ebase, EPW)], ia_v, sa1).wait()
    pltpu.make_async_copy(ib_hbm.at[pl.ds(ebase, EPW)], ib_v, sb1).wait()
    plsc.subcore_barrier()
    issue(0, ra0, rb0, sa0, sb0)

    def pair(i, _):
        c = 2 * i
        issue(c + 1, ra1, rb1, sa1, sb1)
        wait(c, ra0, rb0, sa0, sb0)
        compute(ra0, rb0, c)
        issue(c + 2, ra0, rb0, sa0, sb0)
        wait(c + 1, ra1, rb1, sa1, sb1)
        compute(ra1, rb1, c + 1)
        return 0

    lax.fori_loop(0, PAIRS, pair, 0)
    wait(NCHUNK - 1, ra0, rb0, sa0, sb0)
    compute(ra0, rb0, NCHUNK - 1)
    pltpu.sync_copy(out_v, out_hbm.at[pl.ds(ebase, EPW)])


def kernel(x_author, edge_label_index):
    idx = edge_label_index.astype(jnp.int32)
    return _edge_dot(x_author, idx[0], idx[1])
